# baseline (speedup 1.0000x reference)
import jax
import jax.numpy as jnp
from jax import lax
from jax.experimental import pallas as pl
from jax.experimental.pallas import tpu as pltpu
from jax.experimental.pallas import tpu_sc as plsc

B = 16384
D = 128
LANES = 16
NUM_WORKERS = 32
ROWS_PER_WORKER = B // NUM_WORKERS
CHUNK = 128
NCHUNKS = ROWS_PER_WORKER // CHUNK
NBUF = 3


def _body(z_hbm, y_hbm, w_hbm, out_hbm, idx_all, rows, zb, sem_g, sem_z, sem_o):
    wid = lax.axis_index("s") * 2 + lax.axis_index("c")
    base = wid * ROWS_PER_WORKER
    pltpu.sync_copy(y_hbm.at[wid], idx_all)

    def start_chunk(ck, p):
        g = pltpu.async_copy(w_hbm.at[idx_all.at[ck]], rows.at[p], sem_g.at[p])
        zc = pltpu.async_copy(
            z_hbm.at[pl.ds(base + ck * CHUNK, CHUNK)], zb.at[p], sem_z.at[p]
        )
        return g, zc

    inflight = {ck: start_chunk(ck, ck % NBUF) for ck in range(NBUF)}
    out_cps = {}
    for ck in range(NCHUNKS):
        p = ck % NBUF
        g, zc = inflight[ck]
        g.wait()
        zc.wait()

        @plsc.parallel_loop(0, CHUNK, unroll=2)
        def add_row(r):
            for c in range(D // LANES):
                s = pl.ds(c * LANES, LANES)
                plsc.addupdate(zb.at[p, r, s], rows[p, r, s])

        out_cps[ck] = pltpu.async_copy(
            zb.at[p], out_hbm.at[pl.ds(base + ck * CHUNK, CHUNK)], sem_o.at[p]
        )
        nxt = ck + 1
        if NBUF <= nxt < NCHUNKS:
            out_cps[nxt - NBUF].wait()
            inflight[nxt] = start_chunk(nxt, nxt % NBUF)
    for ck in range(max(1, NCHUNKS - NBUF), NCHUNKS):
        out_cps[ck].wait()


@jax.jit
def _run(z, y, embedding_weight):
    mesh = plsc.VectorSubcoreMesh(core_axis_name="c", subcore_axis_name="s")
    return pl.kernel(
        _body,
        out_type=jax.ShapeDtypeStruct((B, D), jnp.float32),
        mesh=mesh,
        scratch_types=[
            pltpu.VMEM((NCHUNKS, CHUNK), jnp.int32),
            pltpu.VMEM((NBUF, CHUNK, D), jnp.float32),
            pltpu.VMEM((NBUF, CHUNK, D), jnp.float32),
            pltpu.SemaphoreType.DMA((NBUF,)),
            pltpu.SemaphoreType.DMA((NBUF,)),
            pltpu.SemaphoreType.DMA((NBUF,)),
        ],
    )(z, y.reshape(NUM_WORKERS, NCHUNKS, CHUNK), embedding_weight)


def kernel(z, y, embedding_weight):
    return _run(z, y.astype(jnp.int32), embedding_weight)


# speedup vs baseline: 1.0015x; 1.0015x over previous
import jax
import jax.numpy as jnp
from jax import lax
from jax.experimental import pallas as pl
from jax.experimental.pallas import tpu as pltpu
from jax.experimental.pallas import tpu_sc as plsc

B = 16384
D = 128
LANES = 16
NUM_WORKERS = 32
ROWS_PER_WORKER = B // NUM_WORKERS
CHUNK = 128
NCHUNKS = ROWS_PER_WORKER // CHUNK
NBUF = 3


def _body(z_hbm, y_hbm, w_hbm, out_hbm, idx_all, rows, zb, sem_g, sem_z, sem_o):
    wid = lax.axis_index("s") * 2 + lax.axis_index("c")
    base = wid * ROWS_PER_WORKER
    pltpu.sync_copy(y_hbm.at[wid], idx_all)

    gathers = [
        pltpu.async_copy(w_hbm.at[idx_all.at[ck]], rows.at[ck], sem_g.at[ck])
        for ck in range(NCHUNKS)
    ]

    def start_z(ck):
        return pltpu.async_copy(
            z_hbm.at[pl.ds(base + ck * CHUNK, CHUNK)],
            zb.at[ck % NBUF],
            sem_z.at[ck % NBUF],
        )

    zcps = {ck: start_z(ck) for ck in range(NBUF)}
    out_cps = {}
    waited = set()
    for ck in range(NCHUNKS):
        p = ck % NBUF
        gathers[ck].wait()
        zcps[ck].wait()

        @plsc.parallel_loop(0, CHUNK, unroll=2)
        def add_row(r):
            for c in range(D // LANES):
                s = pl.ds(c * LANES, LANES)
                plsc.addupdate(zb.at[p, r, s], rows[ck, r, s])

        out_cps[ck] = pltpu.async_copy(
            zb.at[p], out_hbm.at[pl.ds(base + ck * CHUNK, CHUNK)], sem_o.at[p]
        )
        nxt = ck + NBUF - 1
        if NBUF <= nxt < NCHUNKS and nxt not in zcps:
            out_cps[nxt - NBUF].wait()
            waited.add(nxt - NBUF)
            zcps[nxt] = start_z(nxt)
    for ck in range(NCHUNKS):
        if ck not in waited:
            out_cps[ck].wait()


@jax.jit
def _run(z, y, embedding_weight):
    mesh = plsc.VectorSubcoreMesh(core_axis_name="c", subcore_axis_name="s")
    return pl.kernel(
        _body,
        out_type=jax.ShapeDtypeStruct((B, D), jnp.float32),
        mesh=mesh,
        scratch_types=[
            pltpu.VMEM((NCHUNKS, CHUNK), jnp.int32),
            pltpu.VMEM((NCHUNKS, CHUNK, D), jnp.float32),
            pltpu.VMEM((NBUF, CHUNK, D), jnp.float32),
            pltpu.SemaphoreType.DMA((NCHUNKS,)),
            pltpu.SemaphoreType.DMA((NBUF,)),
            pltpu.SemaphoreType.DMA((NBUF,)),
        ],
    )(z, y.reshape(NUM_WORKERS, NCHUNKS, CHUNK), embedding_weight)


def kernel(z, y, embedding_weight):
    return _run(z, y.astype(jnp.int32), embedding_weight)


# speedup vs baseline: 1.0460x; 1.0445x over previous
import jax
import jax.numpy as jnp
from jax import lax
from jax.experimental import pallas as pl
from jax.experimental.pallas import tpu as pltpu
from jax.experimental.pallas import tpu_sc as plsc

B = 16384
D = 128
LANES = 16
NUM_WORKERS = 32
ROWS_PER_WORKER = B // NUM_WORKERS
CHUNK = 128
NCHUNKS = ROWS_PER_WORKER // CHUNK
NBUF = 3


def _body(z_hbm, y_hbm, w_hbm, out_hbm, idx_all, zb, sem_z, sem_g, sem_o):
    wid = lax.axis_index("s") * 2 + lax.axis_index("c")
    base = wid * ROWS_PER_WORKER
    pltpu.sync_copy(y_hbm.at[wid], idx_all)

    def start_z(ck):
        return pltpu.async_copy(
            z_hbm.at[pl.ds(base + ck * CHUNK, CHUNK)],
            zb.at[ck % NBUF],
            sem_z.at[ck % NBUF],
        )

    def start_gather_add(ck):
        return pltpu.async_copy(
            w_hbm.at[idx_all.at[ck]], zb.at[ck % NBUF], sem_g.at[ck % NBUF],
            add=True,
        )

    zcps = {ck: start_z(ck) for ck in range(NBUF)}
    gcps = {}
    out_cps = {}
    waited = set()
    for ck in range(NCHUNKS):
        p = ck % NBUF
        zcps[ck].wait()
        gcps[ck] = start_gather_add(ck)
        gcps[ck].wait()
        out_cps[ck] = pltpu.async_copy(
            zb.at[p], out_hbm.at[pl.ds(base + ck * CHUNK, CHUNK)], sem_o.at[p]
        )
        nxt = ck + NBUF
        if nxt < NCHUNKS:
            out_cps[ck].wait()
            waited.add(ck)
            zcps[nxt] = start_z(nxt)
    for ck in range(NCHUNKS):
        if ck not in waited:
            out_cps[ck].wait()


@jax.jit
def _run(z, y, embedding_weight):
    mesh = plsc.VectorSubcoreMesh(core_axis_name="c", subcore_axis_name="s")
    return pl.kernel(
        _body,
        out_type=jax.ShapeDtypeStruct((B, D), jnp.float32),
        mesh=mesh,
        scratch_types=[
            pltpu.VMEM((NCHUNKS, CHUNK), jnp.int32),
            pltpu.VMEM((NBUF, CHUNK, D), jnp.float32),
            pltpu.SemaphoreType.DMA((NBUF,)),
            pltpu.SemaphoreType.DMA((NBUF,)),
            pltpu.SemaphoreType.DMA((NBUF,)),
        ],
    )(z, y.reshape(NUM_WORKERS, NCHUNKS, CHUNK), embedding_weight)


def kernel(z, y, embedding_weight):
    return _run(z, y.astype(jnp.int32), embedding_weight)


# speedup vs baseline: 1.0503x; 1.0041x over previous
import jax
import jax.numpy as jnp
from jax import lax
from jax.experimental import pallas as pl
from jax.experimental.pallas import tpu as pltpu
from jax.experimental.pallas import tpu_sc as plsc

B = 16384
D = 128
LANES = 16
NUM_WORKERS = 32
ROWS_PER_WORKER = B // NUM_WORKERS
CHUNK = 128
NCHUNKS = ROWS_PER_WORKER // CHUNK
NBUF = 3


def _body(z_hbm, y_hbm, w_hbm, out_hbm, idx_all, rows, zb, sem_g, sem_z, sem_o):
    wid = lax.axis_index("s") * 2 + lax.axis_index("c")
    base = wid * ROWS_PER_WORKER
    pltpu.sync_copy(y_hbm.at[wid], idx_all)

    def start_gather(ck):
        p = ck % NBUF
        return pltpu.async_copy(w_hbm.at[idx_all.at[ck]], rows.at[p], sem_g.at[p])

    def start_z(ck):
        p = ck % NBUF
        return pltpu.async_copy(
            z_hbm.at[pl.ds(base + ck * CHUNK, CHUNK)], zb.at[p], sem_z.at[p]
        )

    gcps = {ck: start_gather(ck) for ck in range(NBUF)}
    zcps = {ck: start_z(ck) for ck in range(NBUF)}
    out_cps = {}
    waited = set()
    for ck in range(NCHUNKS):
        p = ck % NBUF
        if ck + NBUF - 1 < NCHUNKS and ck - 1 >= 0:
            out_cps[ck - 1].wait()
            waited.add(ck - 1)
            zcps[ck + NBUF - 1] = start_z(ck + NBUF - 1)
        gcps[ck].wait()
        zcps[ck].wait()

        @plsc.parallel_loop(0, CHUNK, unroll=2)
        def add_row(r):
            for c in range(D // LANES):
                s = pl.ds(c * LANES, LANES)
                plsc.addupdate(zb.at[p, r, s], rows[p, r, s])

        nxt = ck + NBUF
        if nxt < NCHUNKS:
            gcps[nxt] = start_gather(nxt)
        out_cps[ck] = pltpu.async_copy(
            zb.at[p], out_hbm.at[pl.ds(base + ck * CHUNK, CHUNK)], sem_o.at[p]
        )
    for ck in range(NCHUNKS):
        if ck not in waited:
            out_cps[ck].wait()


@jax.jit
def _run(z, y, embedding_weight):
    mesh = plsc.VectorSubcoreMesh(core_axis_name="c", subcore_axis_name="s")
    return pl.kernel(
        _body,
        out_type=jax.ShapeDtypeStruct((B, D), jnp.float32),
        mesh=mesh,
        scratch_types=[
            pltpu.VMEM((NCHUNKS, CHUNK), jnp.int32),
            pltpu.VMEM((NBUF, CHUNK, D), jnp.float32),
            pltpu.VMEM((NBUF, CHUNK, D), jnp.float32),
            pltpu.SemaphoreType.DMA((NBUF,)),
            pltpu.SemaphoreType.DMA((NBUF,)),
            pltpu.SemaphoreType.DMA((NBUF,)),
        ],
    )(z, y.reshape(NUM_WORKERS, NCHUNKS, CHUNK), embedding_weight)


def kernel(z, y, embedding_weight):
    return _run(z, y.astype(jnp.int32), embedding_weight)
